# Initial kernel scaffold; baseline (speedup 1.0000x reference)
import jax
import jax.numpy as jnp
from jax.experimental import pallas as pl


def kernel(x_nchw, w1, b1, w2, b2, fc1w3, fc1b, fc2w, fc2b):
    raise NotImplementedError("write your pallas kernel here")


# speedup vs baseline: 4.4982x; 4.4982x over previous
import functools

import jax
import jax.numpy as jnp
import numpy as np
from jax import lax
from jax.experimental import pallas as pl
from jax.experimental.pallas import tpu as pltpu

_VMEM_LIMIT = 48 * 1024 * 1024

_SIGMA = [0, 2, 4, 6, 8, 10, 12, 14, 1, 3, 5, 7, 9, 11, 13, 15]
_TAU = tuple([2 * s for s in _SIGMA] + [2 * s + 1 for s in _SIGMA])


def _colperm(W, C):
    idx = np.arange(W * C).reshape(W, C)
    return np.concatenate([idx[0::2].ravel(), idx[1::2].ravel()])

_PERM1 = tuple(_colperm(32, 32))
_PERM2 = tuple(_colperm(16, 64))


def _shift_down(x, Bn, R, L):
    m = lax.broadcasted_iota(jnp.int32, (Bn, R, L), 1) == 0
    return jnp.where(m, 0.0, pltpu.roll(x.reshape(Bn * R, L), 1, axis=0)
                     .reshape(Bn, R, L))


def _shift_up(x, Bn, R, L):
    m = lax.broadcasted_iota(jnp.int32, (Bn, R, L), 1) == R - 1
    return jnp.where(m, 0.0, pltpu.roll(x.reshape(Bn * R, L), Bn * R - 1,
                                        axis=0).reshape(Bn, R, L))


def _fused_cnn_kernel(x_ref, w1_ref, b1_ref, w2_ref, b2_ref, fc1w_ref,
                      fc1b_ref, fc2w_ref, fc2b_ref, o_ref, *, B):
    x3 = x_ref[...]
    A = x3[:, :16]
    Bh = x3[:, 16:]
    A_lo, A_hi = A[:, :8], A[:, 8:]
    B_lo, B_hi = Bh[:, :8], Bh[:, 8:]
    xd_A = jnp.concatenate([_shift_down(B_hi, B, 8, 128), B_lo], axis=1)
    xu_B = jnp.concatenate([A_hi, _shift_up(A_lo, B, 8, 128)], axis=1)
    xcat = jnp.concatenate(
        [jnp.concatenate([xd_A, A, Bh], axis=-1),
         jnp.concatenate([A, Bh, xu_B], axis=-1)], axis=1)

    acc1 = jnp.dot(xcat.astype(jnp.bfloat16).reshape(B * 32, 384), w1_ref[...],
                   preferred_element_type=jnp.float32).reshape(B, 32, 1024)
    v = jnp.maximum(acc1[:, :16], acc1[:, 16:])
    v = jnp.maximum(v + b1_ref[0, :], 0.0)
    p1 = jnp.maximum(v[:, :, :512], v[:, :, 512:])

    E, O = p1[:, :8], p1[:, 8:]
    pd_E = _shift_down(O, B, 8, 512)
    pu_O = _shift_up(E, B, 8, 512)
    pcat = jnp.concatenate(
        [jnp.concatenate([pd_E, E, O], axis=-1),
         jnp.concatenate([E, O, pu_O], axis=-1)], axis=1)

    acc2 = jnp.dot(pcat.astype(jnp.bfloat16).reshape(B * 16, 1536), w2_ref[...],
                   preferred_element_type=jnp.float32).reshape(B, 16, 1024)
    v2 = jnp.maximum(acc2[:, :8], acc2[:, 8:])
    v2 = jnp.maximum(v2 + b2_ref[0, :], 0.0)
    p2 = jnp.maximum(v2[:, :, :512], v2[:, :, 512:])

    q = jnp.concatenate([p2[:, ho, :] for ho in range(8)],
                        axis=-1).astype(jnp.bfloat16)
    acc = jnp.dot(q, fc1w_ref[...], preferred_element_type=jnp.float32)
    h = jnp.maximum(acc + fc1b_ref[0, :], 0.0).astype(jnp.bfloat16)

    out = jnp.dot(h, fc2w_ref[...],
                  preferred_element_type=jnp.float32) + fc2b_ref[0, :]
    o_ref[...] = out.astype(o_ref.dtype)


def _pick_batch_tile(N):
    Np = ((N + 7) // 8) * 8
    for b in (64, 32, 24, 16, 8):
        if Np % b == 0 and Np // b >= 2:
            return b
    for b in (64, 32, 24, 16, 8):
        if Np % b == 0:
            return b
    return 8


@functools.partial(jax.jit, static_argnames=("block_n",))
def _forward(x_nchw, w1, b1, w2, b2, fc1w3, fc1b, fc2w, fc2b, block_n):
    N = x_nchw.shape[0]
    assert x_nchw.shape[1:] == (3, 32, 32)
    B = block_n
    Np = ((N + B - 1) // B) * B

    tau = jnp.array(_TAU, jnp.int32)
    perm1 = jnp.array(_PERM1, jnp.int32)
    perm2 = jnp.array(_PERM2, jnp.int32)

    x = jnp.transpose(x_nchw, (0, 2, 3, 1))[:, tau].reshape(N, 32, 96)
    x = jnp.pad(x, ((0, Np - N), (0, 0), (0, 32)))

    w1p = jnp.concatenate([w1[0], w1[1], w1[2]], axis=0)[:, perm1]
    w2p = jnp.concatenate([w2[0], w2[1], w2[2]], axis=0)[:, perm2]
    w1p = w1p.astype(jnp.bfloat16)
    w2p = w2p.astype(jnp.bfloat16)
    b1p = b1[:, perm1]
    b2p = b2[:, perm2]
    fc1w = fc1w3.reshape(4096, 128).astype(jnp.bfloat16)
    fc2wb = fc2w.astype(jnp.bfloat16)

    kern = functools.partial(_fused_cnn_kernel, B=B)
    out = pl.pallas_call(
        kern,
        out_shape=jax.ShapeDtypeStruct((Np, 128), jnp.float32),
        grid=(Np // B,),
        in_specs=[
            pl.BlockSpec((B, 32, 128), lambda n: (n, 0, 0)),
            pl.BlockSpec((384, 1024), lambda n: (0, 0)),
            pl.BlockSpec((1, 1024), lambda n: (0, 0)),
            pl.BlockSpec((1536, 1024), lambda n: (0, 0)),
            pl.BlockSpec((1, 1024), lambda n: (0, 0)),
            pl.BlockSpec((4096, 128), lambda n: (0, 0)),
            pl.BlockSpec((1, 128), lambda n: (0, 0)),
            pl.BlockSpec((128, 128), lambda n: (0, 0)),
            pl.BlockSpec((1, 128), lambda n: (0, 0)),
        ],
        out_specs=pl.BlockSpec((B, 128), lambda n: (n, 0)),
        compiler_params=pltpu.CompilerParams(
            dimension_semantics=("parallel",),
            vmem_limit_bytes=_VMEM_LIMIT,
        ),
    )(x, w1p, b1p, w2p, b2p, fc1w, fc1b, fc2wb, fc2b)
    return out[:N, :10]


def kernel(x_nchw, w1, b1, w2, b2, fc1w3, fc1b, fc2w, fc2b):
    return _forward(x_nchw, w1, b1, w2, b2, fc1w3, fc1b, fc2w, fc2b,
                    _pick_batch_tile(x_nchw.shape[0]))
